# baseline (speedup 1.0000x reference)
import functools

import jax
import jax.numpy as jnp
from jax import lax
from jax.experimental import pallas as pl
from jax.experimental.pallas import tpu as pltpu
from jax.experimental.pallas import tpu_sc as plsc

_B, _C, _N = 8, 256, 16384
_M = 4096
_L = 16
_NC, _NS = 2, 16
_NW = _NC * _NS
_R = _B * _C
_RPW = _R // _NW
_NBUF = 4
_FBUF = 6


@functools.partial(
    pl.kernel,
    out_type=jax.ShapeDtypeStruct((_R, _M), jnp.float32),
    mesh=plsc.VectorSubcoreMesh(core_axis_name="c", subcore_axis_name="s"),
    compiler_params=pltpu.CompilerParams(needs_layout_passes=False),
    scratch_types=[
        pltpu.VMEM((_M,), jnp.int32),
        pltpu.VMEM((_N,), jnp.float32),
        pltpu.VMEM((_N,), jnp.float32),
        pltpu.VMEM((_N,), jnp.float32),
        pltpu.VMEM((_N,), jnp.float32),
        pltpu.VMEM((_N,), jnp.float32),
        pltpu.VMEM((_N,), jnp.float32),
        pltpu.VMEM((_M,), jnp.float32),
        pltpu.VMEM((_M,), jnp.float32),
        pltpu.VMEM((_M,), jnp.float32),
        pltpu.VMEM((_M,), jnp.float32),
        pltpu.SemaphoreType.DMA,
        pltpu.SemaphoreType.DMA,
        pltpu.SemaphoreType.DMA,
        pltpu.SemaphoreType.DMA,
        pltpu.SemaphoreType.DMA,
        pltpu.SemaphoreType.DMA,
        pltpu.SemaphoreType.DMA,
        pltpu.SemaphoreType.DMA,
        pltpu.SemaphoreType.DMA,
        pltpu.SemaphoreType.DMA,
    ],
)
def _gather_rows(feat_hbm, idx_hbm, out_hbm, idx_v, fv0, fv1, fv2, fv3,
                 fv4, fv5, ov0, ov1, ov2, ov3, fs0, fs1, fs2, fs3, fs4, fs5,
                 os0, os1, os2, os3):
    fv = (fv0, fv1, fv2, fv3, fv4, fv5)
    ov = (ov0, ov1, ov2, ov3)
    fsem = (fs0, fs1, fs2, fs3, fs4, fs5)
    osem = (os0, os1, os2, os3)
    wid = lax.axis_index("s") * _NC + lax.axis_index("c")
    base = wid * _RPW
    pltpu.sync_copy(idx_hbm.at[base // _C], idx_v)

    for k in range(_FBUF):
        pltpu.async_copy(feat_hbm.at[base + k], fv[k], fsem[k])

    def do_row(i, kf, ko, guard_out, guard_pref):
        r = base + i
        pltpu.make_async_copy(feat_hbm.at[r], fv[kf], fsem[kf]).wait()

        if guard_out:
            @pl.when(r >= base + _NBUF)
            def _wait_out():
                pltpu.make_async_copy(ov[ko], out_hbm.at[r], osem[ko]).wait()
        else:
            pltpu.make_async_copy(ov[ko], out_hbm.at[r], osem[ko]).wait()

        @plsc.parallel_loop(0, _M, step=_L, unroll=8)
        def _gather(j):
            iv = idx_v[pl.ds(j, _L)]
            ov[ko][pl.ds(j, _L)] = plsc.load_gather(fv[kf], [iv])

        pltpu.async_copy(ov[ko], out_hbm.at[r], osem[ko])

        if guard_pref:
            @pl.when(r + _FBUF < base + _RPW)
            def _prefetch():
                pltpu.async_copy(feat_hbm.at[r + _FBUF], fv[kf], fsem[kf])
        else:
            pltpu.async_copy(feat_hbm.at[r + _FBUF], fv[kf], fsem[kf])

    _GRP = 12

    def group_body(g, carry):
        i0 = g * _GRP
        for k in range(_GRP):
            do_row(i0 + k, k % _FBUF, k % _NBUF,
                   guard_out=True, guard_pref=False)
        return carry

    lax.fori_loop(0, 4, group_body, 0)
    for i in range(48, _RPW):
        do_row(i, i % _FBUF, i % _NBUF, guard_out=False, guard_pref=True)

    for k in range(_NBUF):
        pltpu.make_async_copy(ov[k], out_hbm.at[base], osem[k]).wait()


def kernel(features, idx):
    feat2d = features.reshape(_R, _N)
    out2d = _gather_rows(feat2d, idx)
    return out2d.reshape(_B, _C, _M)


# speedup vs baseline: 1.0372x; 1.0372x over previous
import functools

import jax
import jax.numpy as jnp
from jax import lax
from jax.experimental import pallas as pl
from jax.experimental.pallas import tpu as pltpu
from jax.experimental.pallas import tpu_sc as plsc

_B, _C, _N = 8, 256, 16384
_M = 4096
_L = 16
_NC, _NS = 2, 16
_NW = _NC * _NS
_R = _B * _C
_RPW = _R // _NW
_NBUF = 4


@functools.partial(
    pl.kernel,
    out_type=jax.ShapeDtypeStruct((_R, _M), jnp.float32),
    mesh=plsc.VectorSubcoreMesh(core_axis_name="c", subcore_axis_name="s"),
    compiler_params=pltpu.CompilerParams(needs_layout_passes=False),
    scratch_types=[
        pltpu.VMEM((_M,), jnp.int32),
        pltpu.VMEM((_N,), jnp.float32),
        pltpu.VMEM((_N,), jnp.float32),
        pltpu.VMEM((_N,), jnp.float32),
        pltpu.VMEM((_N,), jnp.float32),
        pltpu.VMEM((_M,), jnp.float32),
        pltpu.VMEM((_M,), jnp.float32),
        pltpu.VMEM((_M,), jnp.float32),
        pltpu.VMEM((_M,), jnp.float32),
        pltpu.SemaphoreType.DMA,
        pltpu.SemaphoreType.DMA,
        pltpu.SemaphoreType.DMA,
        pltpu.SemaphoreType.DMA,
        pltpu.SemaphoreType.DMA,
        pltpu.SemaphoreType.DMA,
        pltpu.SemaphoreType.DMA,
        pltpu.SemaphoreType.DMA,
    ],
)
def _gather_rows(feat_hbm, idx_hbm, out_hbm, idx_v, fv0, fv1, fv2, fv3,
                 ov0, ov1, ov2, ov3, fs0, fs1, fs2, fs3, os0, os1, os2, os3):
    fv = (fv0, fv1, fv2, fv3)
    ov = (ov0, ov1, ov2, ov3)
    fsem = (fs0, fs1, fs2, fs3)
    osem = (os0, os1, os2, os3)
    wid = lax.axis_index("s") * _NC + lax.axis_index("c")
    base = wid * _RPW
    pltpu.sync_copy(idx_hbm.at[base // _C], idx_v)

    for k in range(_NBUF):
        pltpu.async_copy(feat_hbm.at[base + k], fv[k], fsem[k])

    def group_body(g, carry):
        i = g * _NBUF
        for k in range(_NBUF):
            r = base + i + k
            pltpu.make_async_copy(feat_hbm.at[r], fv[k], fsem[k]).wait()

            @pl.when(i + k >= _NBUF)
            def _wait_out():
                pltpu.make_async_copy(ov[k], out_hbm.at[r], osem[k]).wait()

            @plsc.parallel_loop(0, _M, step=_L, unroll=8)
            def _gather(j):
                iv = idx_v[pl.ds(j, _L)]
                ov[k][pl.ds(j, _L)] = plsc.load_gather(fv[k], [iv])

            pltpu.async_copy(ov[k], out_hbm.at[r], osem[k])

            @pl.when(i + k + _NBUF < _RPW)
            def _prefetch():
                pltpu.async_copy(feat_hbm.at[r + _NBUF], fv[k], fsem[k])
        return carry

    lax.fori_loop(0, _RPW // _NBUF, group_body, 0)

    for k in range(_NBUF):
        pltpu.make_async_copy(ov[k], out_hbm.at[base], osem[k]).wait()


def kernel(features, idx):
    feat2d = features.reshape(_R, _N)
    out2d = _gather_rows(feat2d, idx)
    return out2d.reshape(_B, _C, _M)
